# baseline (speedup 1.0000x reference)
import jax
import jax.numpy as jnp
from jax import lax
from jax.experimental import pallas as pl
from jax.experimental.pallas import tpu as pltpu
from jax.experimental.pallas import tpu_sc as plsc

_N = 131072
_ZROWS = 86
_D = 256
_L = 16

_NC = 2
_NS = 16
_NW = _NC * _NS
_BPW = _N // _NW
_C = 128
_NCHUNK = _BPW // _C
_TABW = _ZROWS * _D


def _prep_body(ec_ref, w_ref, zt_ref, tab_ref):
    t = (jnp.dot(ec_ref[...], w_ref[...], preferred_element_type=jnp.float32)
         + zt_ref[...])
    tab_ref[...] = jnp.broadcast_to(t[None], (_NW, _ZROWS, _D))


def _prep(ec86, w, zt):
    return pl.pallas_call(
        _prep_body,
        out_shape=jax.ShapeDtypeStruct((_NW, _ZROWS, _D), jnp.float32),
    )(ec86, w, zt)


def _gather_body(table_hbm, idx_hbm, out_hbm,
                 table_v, idx_v, bufs, osems):
    cid = lax.axis_index("c")
    sid = lax.axis_index("s")
    wid = sid * _NC + cid
    base = wid * _BPW

    pltpu.sync_copy(table_hbm.at[pl.ds(wid * _TABW, _TABW)], table_v)
    pltpu.sync_copy(idx_hbm.at[pl.ds(base, _BPW)], idx_v)

    iota16 = jax.lax.iota(jnp.int32, _L)

    def assemble(ci, buf):
        @pl.loop(0, _C)
        def _rows(rr):
            ridx = ci * _C + rr
            zsplat = plsc.load_gather(
                idx_v, [jnp.full((_L,), ridx, jnp.int32)])
            zbase = zsplat * _D + iota16
            dst0 = rr * _D
            for c in range(_D // _L):
                val = plsc.load_gather(table_v, [zbase + (_L * c)])
                buf[pl.ds(dst0 + _L * c, _L)] = val

    def start_scatter(ci, b):
        pltpu.async_copy(bufs[b],
                         out_hbm.at[pl.ds((base + ci * _C) * _D, _C * _D)],
                         osems[b])

    def wait_scatter(b):
        pltpu.make_async_copy(bufs[b],
                              out_hbm.at[pl.ds(base * _D, _C * _D)],
                              osems[b]).wait()

    @pl.loop(0, _NCHUNK, step=2)
    def _chunks(g):
        for b in range(2):
            ci = g + b

            @pl.when(ci >= 2)
            def _():
                wait_scatter(b)
            assemble(ci, bufs[b])
            start_scatter(ci, b)

    wait_scatter(0)
    wait_scatter(1)


def kernel(z, elec_config, m_weight, z_table):
    zi = z.astype(jnp.int32)
    tab = _prep(elec_config[:_ZROWS], m_weight, z_table).reshape(_NW * _TABW)
    mesh = plsc.VectorSubcoreMesh(core_axis_name="c", subcore_axis_name="s",
                                  num_cores=_NC, num_subcores=_NS)
    gather = pl.kernel(
        _gather_body,
        out_type=jax.ShapeDtypeStruct((_N * _D,), jnp.float32),
        mesh=mesh,
        compiler_params=pltpu.CompilerParams(needs_layout_passes=False),
        scratch_types=[
            pltpu.VMEM((_TABW,), jnp.float32),
            pltpu.VMEM((_BPW,), jnp.int32),
            [pltpu.VMEM((_C * _D,), jnp.float32) for _ in range(2)],
            [pltpu.SemaphoreType.DMA for _ in range(2)],
        ],
    )
    return gather(tab, zi).reshape(_N, _D)
